# Initial kernel scaffold; baseline (speedup 1.0000x reference)
import jax
import jax.numpy as jnp
from jax.experimental import pallas as pl


def kernel(hidden_states, weight):
    raise NotImplementedError("write your pallas kernel here")


# speedup vs baseline: 1.8948x; 1.8948x over previous
import jax
import jax.numpy as jnp
from jax import lax
from jax.experimental import pallas as pl
from jax.experimental.pallas import tpu as pltpu

TOP_K = 2
NUM_EXPERTS = 8
DIM = 768
ALPHA = 0.001
BSZ = 4
SEQ = 8192

TOKENS = BSZ * SEQ
BLOCK_T = 2048
GRID = TOKENS // BLOCK_T
BLOCKS_PER_BATCH = SEQ // BLOCK_T


def _gate_body(x_ref, w_ref, idx_ref, tw_ref, cnt_ref, ssum_ref):
    i = pl.program_id(0)
    x = x_ref[...]
    w = w_ref[...]
    logits = lax.dot_general(
        x, w, (((1,), (1,)), ((), ())),
        preferred_element_type=jnp.float32)

    m = jnp.max(logits, axis=1, keepdims=True)
    ex = jnp.exp(logits - m)
    denom = jnp.sum(ex, axis=1, keepdims=True)
    scores = ex / denom

    eidx = lax.broadcasted_iota(jnp.int32, scores.shape, 1)
    m1 = jnp.max(scores, axis=1, keepdims=True)
    i1 = jnp.min(jnp.where(scores == m1, eidx, NUM_EXPERTS), axis=1, keepdims=True)
    masked = jnp.where(eidx == i1, -jnp.inf, scores)
    m2 = jnp.max(masked, axis=1, keepdims=True)
    i2 = jnp.min(jnp.where(masked == m2, eidx, NUM_EXPERTS), axis=1, keepdims=True)

    d = m1 + m2 + 1e-20
    tw_ref[...] = jnp.concatenate([m1 / d, m2 / d], axis=1)
    idx_ref[...] = jnp.concatenate([i1, i2], axis=1)

    onehot = (eidx == i1).astype(jnp.float32) + (eidx == i2).astype(jnp.float32)
    cnt = jnp.sum(onehot, axis=0, keepdims=True)[None]
    ssum = jnp.sum(scores, axis=0, keepdims=True)[None]

    @pl.when(i % BLOCKS_PER_BATCH == 0)
    def _init():
        cnt_ref[...] = cnt
        ssum_ref[...] = ssum

    @pl.when(i % BLOCKS_PER_BATCH != 0)
    def _acc():
        cnt_ref[...] += cnt
        ssum_ref[...] += ssum


@jax.jit
def kernel(hidden_states, weight):
    hs = hidden_states.reshape(TOKENS, DIM)
    out_shapes = (
        jax.ShapeDtypeStruct((TOKENS, TOP_K), jnp.int32),
        jax.ShapeDtypeStruct((TOKENS, TOP_K), jnp.float32),
        jax.ShapeDtypeStruct((BSZ, 1, NUM_EXPERTS), jnp.float32),
        jax.ShapeDtypeStruct((BSZ, 1, NUM_EXPERTS), jnp.float32),
    )
    topk_idx, topk_w, cnt, ssum = pl.pallas_call(
        _gate_body,
        grid=(GRID,),
        in_specs=[
            pl.BlockSpec((BLOCK_T, DIM), lambda i: (i, 0)),
            pl.BlockSpec((NUM_EXPERTS, DIM), lambda i: (0, 0)),
        ],
        out_specs=(
            pl.BlockSpec((BLOCK_T, TOP_K), lambda i: (i, 0)),
            pl.BlockSpec((BLOCK_T, TOP_K), lambda i: (i, 0)),
            pl.BlockSpec((1, 1, NUM_EXPERTS), lambda i: (i // BLOCKS_PER_BATCH, 0, 0)),
            pl.BlockSpec((1, 1, NUM_EXPERTS), lambda i: (i // BLOCKS_PER_BATCH, 0, 0)),
        ),
        out_shape=out_shapes,
    )(hs, weight)

    cnt = cnt[:, 0, :]
    ssum = ssum[:, 0, :]
    ce = cnt * (NUM_EXPERTS / (SEQ * TOP_K))
    smean = ssum * (1.0 / SEQ)
    aux_loss = (ce * smean).sum(axis=1).mean() * ALPHA
    return (topk_idx, topk_w, aux_loss)
